# baseline (speedup 1.0000x reference)
import jax
import jax.numpy as jnp
from jax import lax
from jax.experimental import pallas as pl
from jax.experimental.pallas import tpu as pltpu
from jax.experimental.pallas import tpu_sc as plsc

B, L = 1024, 256
NC, NS, LANES = 2, 16, 16
NW = NC * NS
B_SC = B // 2
B_TC = B - B_SC
ROWS = B_SC // NW
NQ = 1
QROWS = ROWS // NQ
CHUNKS = L // LANES
UNROLL = 2


def _bpmll_body(x_hbm, t_hbm, out_hbm, x_v, t_v, o_v, *sems):
    wid = lax.axis_index("s") * NC + lax.axis_index("c")

    copies = []
    for q in range(NQ):
        sl = pl.ds(q * QROWS, QROWS)
        copies.append(pltpu.async_copy(x_hbm.at[wid, sl], x_v.at[sl], sems[2 * q]))
        copies.append(pltpu.async_copy(t_hbm.at[wid, sl], t_v.at[sl], sems[2 * q + 1]))

    zero = jnp.zeros((LANES,), jnp.float32)
    lanes = lax.iota(jnp.int32, LANES)
    m15 = lanes == (LANES - 1)
    lden = jnp.full((LANES,), float(L), jnp.float32)

    def sample_body(r, acc):
        tot, ep, npos = zero, zero, zero
        for c in range(CHUNKS):
            xv = x_v[r, pl.ds(c * LANES, LANES)]
            tv = t_v[r, pl.ds(c * LANES, LANES)]
            z = plsc.bitcast(
                plsc.bitcast(xv, jnp.int32) ^ (tv << 31), jnp.float32)
            e = jnp.exp(z)
            tf = tv.astype(jnp.float32)
            tot = tot + e
            ep = ep + e * tf
            npos = npos + tf
        tot_c = plsc.cumsum(tot)
        ep_c = plsc.cumsum(ep)
        np_c = plsc.cumsum(npos)
        loss = (tot_c - ep_c) * ep_c / (np_c * (lden - np_c))
        return acc + jnp.where(m15, loss, zero)

    acc = zero
    for q in range(NQ):
        copies[2 * q].wait()
        copies[2 * q + 1].wait()
        acc = lax.fori_loop(
            q * QROWS, (q + 1) * QROWS, sample_body, acc, unroll=UNROLL)
    o_v[...] = acc
    pltpu.sync_copy(o_v, out_hbm.at[pl.ds(wid * LANES, LANES)])


def _bpmll_tc_body(x_ref, t_ref, o_ref):
    x = x_ref[...]
    t = t_ref[...]
    tf = t.astype(jnp.float32)
    e = jnp.exp(jnp.where(t == 1, -x, x))
    tot = jnp.sum(e, axis=1)
    ep = jnp.sum(e * tf, axis=1)
    npos = jnp.sum(tf, axis=1)
    loss = (tot - ep) * ep / (npos * (float(L) - npos))
    o_ref[...] = jnp.sum(loss).reshape(1, 1)


_sc_fn = None
_tc_fn = None


def _get_sc_fn():
    global _sc_fn, _tc_fn
    if _sc_fn is None:
        mesh = plsc.VectorSubcoreMesh(
            core_axis_name="c", subcore_axis_name="s", num_cores=NC, num_subcores=NS
        )
        _sc_fn = pl.kernel(
            _bpmll_body,
            out_type=jax.ShapeDtypeStruct((NW * LANES,), jnp.float32),
            mesh=mesh,
            scratch_types=[
                pltpu.VMEM((ROWS, L), jnp.float32),
                pltpu.VMEM((ROWS, L), jnp.int32),
                pltpu.VMEM((LANES,), jnp.float32),
            ] + [pltpu.SemaphoreType.DMA] * (2 * NQ),
            compiler_params=pltpu.CompilerParams(needs_layout_passes=False),
        )
        _tc_fn = pl.pallas_call(
            _bpmll_tc_body,
            grid=(1,),
            in_specs=[
                pl.BlockSpec((B_TC, L), lambda i: (1, 0)),
                pl.BlockSpec((B_TC, L), lambda i: (1, 0)),
            ],
            out_specs=pl.BlockSpec((1, 1), lambda i: (0, 0)),
            out_shape=jax.ShapeDtypeStruct((1, 1), jnp.float32),
        )
    return _sc_fn, _tc_fn


def kernel(input, target):
    sc_fn, tc_fn = _get_sc_fn()
    ti = target.astype(jnp.int32)
    partials = sc_fn(input.reshape(B // ROWS, ROWS, L),
                     ti.reshape(B // ROWS, ROWS, L))
    tc_sum = tc_fn(input, ti)
    return jnp.sum(partials) + tc_sum[0, 0]


# speedup vs baseline: 1.0408x; 1.0408x over previous
import jax
import jax.numpy as jnp
from jax import lax
from jax.experimental import pallas as pl
from jax.experimental.pallas import tpu as pltpu
from jax.experimental.pallas import tpu_sc as plsc

B, L = 1024, 256
NC, NS, LANES = 2, 16, 16
NW = NC * NS
ROWS = B // NW
NQ = 1
QROWS = ROWS // NQ
CHUNKS = L // LANES
UNROLL = 1


def _bpmll_body(x_hbm, t_hbm, out_hbm, x_v, t_v, o_v, *sems):
    wid = lax.axis_index("s") * NC + lax.axis_index("c")

    copies = []
    for q in range(NQ):
        sl = pl.ds(q * QROWS, QROWS)
        copies.append(pltpu.async_copy(x_hbm.at[wid, sl], x_v.at[sl], sems[2 * q]))
        copies.append(pltpu.async_copy(t_hbm.at[wid, sl], t_v.at[sl], sems[2 * q + 1]))

    zero = jnp.zeros((LANES,), jnp.float32)
    lanes = lax.iota(jnp.int32, LANES)
    m15 = lanes == (LANES - 1)
    lden = jnp.full((LANES,), float(L), jnp.float32)

    def sample_body(r, acc):
        tot, ep, npos = zero, zero, zero
        for c in range(CHUNKS):
            xv = x_v[r, pl.ds(c * LANES, LANES)]
            tv = t_v[r, pl.ds(c * LANES, LANES)]
            z = plsc.bitcast(
                plsc.bitcast(xv, jnp.int32) ^ (tv << 31), jnp.float32)
            e = jnp.exp(z)
            tf = tv.astype(jnp.float32)
            tot = tot + e
            ep = ep + e * tf
            npos = npos + tf
        tot_c = plsc.cumsum(tot)
        ep_c = plsc.cumsum(ep)
        np_c = plsc.cumsum(npos)
        loss = (tot_c - ep_c) * ep_c / (np_c * (lden - np_c))
        return acc + jnp.where(m15, loss, zero)

    acc = zero
    for q in range(NQ):
        copies[2 * q].wait()
        copies[2 * q + 1].wait()
        acc = lax.fori_loop(
            q * QROWS, (q + 1) * QROWS, sample_body, acc, unroll=UNROLL)
    o_v[...] = acc
    pltpu.sync_copy(o_v, out_hbm.at[pl.ds(wid * LANES, LANES)])


_sc_fn = None


def _get_sc_fn():
    global _sc_fn
    if _sc_fn is None:
        mesh = plsc.VectorSubcoreMesh(
            core_axis_name="c", subcore_axis_name="s", num_cores=NC, num_subcores=NS
        )
        _sc_fn = pl.kernel(
            _bpmll_body,
            out_type=jax.ShapeDtypeStruct((NW * LANES,), jnp.float32),
            mesh=mesh,
            scratch_types=[
                pltpu.VMEM((ROWS, L), jnp.float32),
                pltpu.VMEM((ROWS, L), jnp.int32),
                pltpu.VMEM((LANES,), jnp.float32),
            ] + [pltpu.SemaphoreType.DMA] * (2 * NQ),
            compiler_params=pltpu.CompilerParams(needs_layout_passes=False),
        )
    return _sc_fn


def kernel(input, target):
    xr = input.reshape(NW, ROWS, L)
    tr = target.astype(jnp.int32).reshape(NW, ROWS, L)
    partials = _get_sc_fn()(xr, tr)
    return jnp.sum(partials)
